# Initial kernel scaffold; baseline (speedup 1.0000x reference)
import functools

import jax
import jax.numpy as jnp
from jax import lax
from jax.experimental import pallas as pl
from jax.experimental.pallas import tpu as pltpu
from jax.experimental.pallas import tpu_sc as plsc

N = 10000
E = 320000
D = 128

NC = 2
NS = 16
NW = NC * NS
EW = E // NW
CH = 80
NCHUNK = EW // CH

NPAD = 10240
SEG = NPAD // NS

ROWS_T = N // NS

E_T = E // NS
K1_CH = 2000
K1_NCHUNK = E_T // K1_CH

_mesh = plsc.VectorSubcoreMesh(core_axis_name="c", subcore_axis_name="s")


@functools.partial(
    pl.kernel,
    mesh=_mesh,
    out_type=jax.ShapeDtypeStruct((NS, SEG), jnp.float32),
    scratch_types=[
        pltpu.VMEM((K1_CH,), jnp.int32),
        pltpu.VMEM((NPAD,), jnp.float32),
        pltpu.VMEM((SEG,), jnp.float32),
        pltpu.VMEM((SEG,), jnp.float32),
        pltpu.VMEM_SHARED((NS, NPAD), jnp.float32),
    ],
)
def _deg_kernel(dst_hbm, deg_hbm, idx_v, hist_v, seg_v, acc_v, stage_s):
    cid = lax.axis_index("c")
    sid = lax.axis_index("s")
    ones = jnp.ones((16,), jnp.float32)

    @pl.when(cid == 0)
    def _():
        def zero_body(i, _):
            hist_v[pl.ds(i * 16, 16)] = jnp.zeros((16,), jnp.float32)
            return ()
        lax.fori_loop(0, NPAD // 16, zero_body, ())

        def chunk_body(j, _):
            base = sid * E_T + j * K1_CH
            pltpu.sync_copy(dst_hbm.at[pl.ds(base, K1_CH)], idx_v)

            def vec_body(k, _):
                idx = idx_v[pl.ds(k * 16, 16)]
                plsc.addupdate_scatter(hist_v, [idx], ones)
                return ()
            lax.fori_loop(0, K1_CH // 16, vec_body, ())
            return ()
        lax.fori_loop(0, K1_NCHUNK, chunk_body, ())

        pltpu.sync_copy(hist_v, stage_s.at[sid])

    plsc.subcore_barrier()

    @pl.when(cid == 0)
    def _():
        def rzero(i, _):
            acc_v[pl.ds(i * 16, 16)] = jnp.zeros((16,), jnp.float32)
            return ()
        lax.fori_loop(0, SEG // 16, rzero, ())

        def radd(j, _):
            pltpu.sync_copy(stage_s.at[j, pl.ds(sid * SEG, SEG)], seg_v)

            def vadd(k, _):
                acc_v[pl.ds(k * 16, 16)] = acc_v[pl.ds(k * 16, 16)] + seg_v[pl.ds(k * 16, 16)]
                return ()
            lax.fori_loop(0, SEG // 16, vadd, ())
            return ()
        lax.fori_loop(0, NS, radd, ())

        pltpu.sync_copy(acc_v, deg_hbm.at[sid])


@functools.partial(
    pl.kernel,
    mesh=_mesh,
    out_type=(
        jax.ShapeDtypeStruct((N, D), jnp.float32),
        jax.ShapeDtypeStruct((N, D), jnp.float32),
    ),
    scratch_types=[
        pltpu.VMEM((NCHUNK, CH), jnp.int32),
        pltpu.VMEM((NCHUNK, CH), jnp.int32),
        pltpu.VMEM((CH, D), jnp.float32),
        pltpu.VMEM_SHARED((N, D), jnp.float32),
        pltpu.SemaphoreType.DMA,
    ],
)
def _msg_kernel(g_hbm, src_hbm, dst_hbm, zeros_hbm, p0_hbm, p1_hbm,
                src_v, dst_v, rows_v, acc_s, sem):
    cid = lax.axis_index("c")
    sid = lax.axis_index("s")
    w = sid * NC + cid

    pltpu.sync_copy(src_hbm.at[w], src_v)
    pltpu.sync_copy(dst_hbm.at[w], dst_v)

    base = sid * ROWS_T

    @pl.when(cid == 0)
    def _():
        pltpu.sync_copy(g_hbm.at[pl.ds(base, ROWS_T)], acc_s.at[pl.ds(base, ROWS_T)])

    @pl.when(cid != 0)
    def _():
        pltpu.sync_copy(zeros_hbm.at[pl.ds(base, ROWS_T)], acc_s.at[pl.ds(base, ROWS_T)])

    plsc.subcore_barrier()

    def chunk_body(c, _):
        pltpu.async_copy(g_hbm.at[src_v.at[c]], rows_v, sem).wait()
        pltpu.sync_copy(rows_v, acc_s.at[dst_v.at[c]], add=True)
        return ()
    lax.fori_loop(0, NCHUNK, chunk_body, ())

    plsc.subcore_barrier()

    @pl.when(cid == 0)
    def _():
        pltpu.sync_copy(acc_s.at[pl.ds(base, ROWS_T)], p0_hbm.at[pl.ds(base, ROWS_T)])

    @pl.when(cid != 0)
    def _():
        pltpu.sync_copy(acc_s.at[pl.ds(base, ROWS_T)], p1_hbm.at[pl.ds(base, ROWS_T)])


def _transform_body(x_ref, w_ref, wc_ref, deg_ref, g_ref, dinv_ref):
    x0 = jnp.dot(x_ref[...], w_ref[...], preferred_element_type=jnp.float32)
    h = jnp.dot(x0, wc_ref[...], preferred_element_type=jnp.float32)
    dinv = lax.rsqrt(deg_ref[...] + 1.0)
    g_ref[...] = h * dinv
    dinv_ref[...] = dinv


def _epilogue_body(p0_ref, p1_ref, dinv_ref, b_ref, out_ref):
    s = (p0_ref[...] + p1_ref[...]) * dinv_ref[...] + b_ref[...]
    out_ref[...] = jnp.where(s >= 0, s, 0.2 * s)


_BLK = 1000


def kernel(x, edge_index, W, Wc, b):
    src = edge_index[0].reshape(NW, NCHUNK, CH)
    dst_flat = edge_index[1]
    dst = dst_flat.reshape(NW, NCHUNK, CH)

    deg2 = _deg_kernel(dst_flat)
    deg = deg2.reshape(NPAD, 1)

    g, dinv = pl.pallas_call(
        _transform_body,
        grid=(N // _BLK,),
        in_specs=[
            pl.BlockSpec((_BLK, D), lambda i: (i, 0)),
            pl.BlockSpec((D, D), lambda i: (0, 0)),
            pl.BlockSpec((D, D), lambda i: (0, 0)),
            pl.BlockSpec((_BLK, 1), lambda i: (i, 0)),
        ],
        out_specs=(
            pl.BlockSpec((_BLK, D), lambda i: (i, 0)),
            pl.BlockSpec((_BLK, 1), lambda i: (i, 0)),
        ),
        out_shape=(
            jax.ShapeDtypeStruct((N, D), jnp.float32),
            jax.ShapeDtypeStruct((N, 1), jnp.float32),
        ),
    )(x, W, Wc, deg)

    zeros = jnp.zeros((N, D), jnp.float32)
    p0, p1 = _msg_kernel(g, src, dst, zeros)

    out = pl.pallas_call(
        _epilogue_body,
        grid=(N // _BLK,),
        in_specs=[
            pl.BlockSpec((_BLK, D), lambda i: (i, 0)),
            pl.BlockSpec((_BLK, D), lambda i: (i, 0)),
            pl.BlockSpec((_BLK, 1), lambda i: (i, 0)),
            pl.BlockSpec((1, D), lambda i: (0, 0)),
        ],
        out_specs=pl.BlockSpec((_BLK, D), lambda i: (i, 0)),
        out_shape=jax.ShapeDtypeStruct((N, D), jnp.float32),
    )(p0, p1, dinv, b.reshape(1, D))

    return out


# speedup vs baseline: 26.3628x; 26.3628x over previous
import functools

import jax
import jax.numpy as jnp
from jax import lax
from jax.experimental import pallas as pl
from jax.experimental.pallas import tpu as pltpu
from jax.experimental.pallas import tpu_sc as plsc

N = 10000
E = 320000
D = 128

NC = 2
NS = 16
NW = NC * NS
EW = E // NW
CH = 80
NCHUNK = EW // CH

NPAD = 10240
SEG = NPAD // NS

ROWS_T = N // NS

E_T = E // NS
K1_CH = 2000
K1_NCHUNK = E_T // K1_CH

_mesh = plsc.VectorSubcoreMesh(core_axis_name="c", subcore_axis_name="s")
_sc_params = pltpu.CompilerParams(
    needs_layout_passes=False, use_tc_tiling_on_sc=False
)


@functools.partial(
    pl.kernel,
    mesh=_mesh,
    out_type=jax.ShapeDtypeStruct((NS, SEG), jnp.float32),
    scratch_types=[
        pltpu.VMEM((K1_CH,), jnp.int32),
        pltpu.VMEM((NPAD,), jnp.float32),
        pltpu.VMEM((SEG,), jnp.float32),
        pltpu.VMEM((SEG,), jnp.float32),
        pltpu.VMEM_SHARED((NS, NPAD), jnp.float32),
    ],
    compiler_params=_sc_params,
)
def _deg_kernel(dst_hbm, deg_hbm, idx_v, hist_v, seg_v, acc_v, stage_s):
    cid = lax.axis_index("c")
    sid = lax.axis_index("s")
    ones = jnp.ones((16,), jnp.float32)

    @pl.when(cid == 0)
    def _():
        def zero_body(i, _):
            hist_v[pl.ds(i * 16, 16)] = jnp.zeros((16,), jnp.float32)
            return ()
        lax.fori_loop(0, NPAD // 16, zero_body, ())

        def chunk_body(j, _):
            base = sid * E_T + j * K1_CH
            pltpu.sync_copy(dst_hbm.at[pl.ds(base, K1_CH)], idx_v)

            def vec_body(k, _):
                idx = idx_v[pl.ds(k * 16, 16)]
                plsc.addupdate_scatter(hist_v, [idx], ones)
                return ()
            lax.fori_loop(0, K1_CH // 16, vec_body, ())
            return ()
        lax.fori_loop(0, K1_NCHUNK, chunk_body, ())

        pltpu.sync_copy(hist_v, stage_s.at[sid])

    plsc.subcore_barrier()

    @pl.when(cid == 0)
    def _():
        def rzero(i, _):
            acc_v[pl.ds(i * 16, 16)] = jnp.zeros((16,), jnp.float32)
            return ()
        lax.fori_loop(0, SEG // 16, rzero, ())

        def radd(j, _):
            pltpu.sync_copy(stage_s.at[j, pl.ds(sid * SEG, SEG)], seg_v)

            def vadd(k, _):
                acc_v[pl.ds(k * 16, 16)] = acc_v[pl.ds(k * 16, 16)] + seg_v[pl.ds(k * 16, 16)]
                return ()
            lax.fori_loop(0, SEG // 16, vadd, ())
            return ()
        lax.fori_loop(0, NS, radd, ())

        pltpu.sync_copy(acc_v, deg_hbm.at[sid])


@functools.partial(
    pl.kernel,
    mesh=_mesh,
    out_type=(
        jax.ShapeDtypeStruct((N, D), jnp.float32),
        jax.ShapeDtypeStruct((N, D), jnp.float32),
    ),
    scratch_types=[
        pltpu.VMEM((NCHUNK, CH), jnp.int32),
        pltpu.VMEM((NCHUNK, CH), jnp.int32),
        pltpu.VMEM((CH, D), jnp.float32),
        pltpu.VMEM_SHARED((N, D), jnp.float32),
        pltpu.SemaphoreType.DMA,
    ],
    compiler_params=_sc_params,
)
def _msg_kernel(g_hbm, src_hbm, dst_hbm, zeros_hbm, p0_hbm, p1_hbm,
                src_v, dst_v, rows_v, acc_s, sem):
    cid = lax.axis_index("c")
    sid = lax.axis_index("s")
    w = sid * NC + cid

    pltpu.sync_copy(src_hbm.at[w], src_v)
    pltpu.sync_copy(dst_hbm.at[w], dst_v)

    base = sid * ROWS_T

    @pl.when(cid == 0)
    def _():
        pltpu.sync_copy(g_hbm.at[pl.ds(base, ROWS_T)], acc_s.at[pl.ds(base, ROWS_T)])

    @pl.when(cid != 0)
    def _():
        pltpu.sync_copy(zeros_hbm.at[pl.ds(base, ROWS_T)], acc_s.at[pl.ds(base, ROWS_T)])

    plsc.subcore_barrier()

    def chunk_body(c, _):
        pltpu.async_copy(g_hbm.at[src_v.at[c]], rows_v, sem).wait()
        pltpu.sync_copy(rows_v, acc_s.at[dst_v.at[c]], add=True)
        return ()
    lax.fori_loop(0, NCHUNK, chunk_body, ())

    plsc.subcore_barrier()

    @pl.when(cid == 0)
    def _():
        pltpu.sync_copy(acc_s.at[pl.ds(base, ROWS_T)], p0_hbm.at[pl.ds(base, ROWS_T)])

    @pl.when(cid != 0)
    def _():
        pltpu.sync_copy(acc_s.at[pl.ds(base, ROWS_T)], p1_hbm.at[pl.ds(base, ROWS_T)])


def _transform_body(x_ref, w_ref, wc_ref, deg_ref, g_ref, dinv_ref):
    x0 = jnp.dot(x_ref[...], w_ref[...], preferred_element_type=jnp.float32)
    h = jnp.dot(x0, wc_ref[...], preferred_element_type=jnp.float32)
    dinv = lax.rsqrt(deg_ref[...] + 1.0)
    g_ref[...] = h * dinv
    dinv_ref[...] = dinv


def _epilogue_body(p0_ref, p1_ref, dinv_ref, b_ref, out_ref):
    s = (p0_ref[...] + p1_ref[...]) * dinv_ref[...] + b_ref[...]
    out_ref[...] = jnp.where(s >= 0, s, 0.2 * s)


_BLK = 1000


def kernel(x, edge_index, W, Wc, b):
    src = edge_index[0].reshape(NW, NCHUNK, CH)
    dst_flat = edge_index[1]
    dst = dst_flat.reshape(NW, NCHUNK, CH)

    deg2 = _deg_kernel(dst_flat)
    deg = deg2.reshape(NPAD, 1)

    g, dinv = pl.pallas_call(
        _transform_body,
        grid=(N // _BLK,),
        in_specs=[
            pl.BlockSpec((_BLK, D), lambda i: (i, 0)),
            pl.BlockSpec((D, D), lambda i: (0, 0)),
            pl.BlockSpec((D, D), lambda i: (0, 0)),
            pl.BlockSpec((_BLK, 1), lambda i: (i, 0)),
        ],
        out_specs=(
            pl.BlockSpec((_BLK, D), lambda i: (i, 0)),
            pl.BlockSpec((_BLK, 1), lambda i: (i, 0)),
        ),
        out_shape=(
            jax.ShapeDtypeStruct((N, D), jnp.float32),
            jax.ShapeDtypeStruct((N, 1), jnp.float32),
        ),
    )(x, W, Wc, deg)

    zeros = jnp.zeros((N, D), jnp.float32)
    p0, p1 = _msg_kernel(g, src, dst, zeros)

    out = pl.pallas_call(
        _epilogue_body,
        grid=(N // _BLK,),
        in_specs=[
            pl.BlockSpec((_BLK, D), lambda i: (i, 0)),
            pl.BlockSpec((_BLK, D), lambda i: (i, 0)),
            pl.BlockSpec((_BLK, 1), lambda i: (i, 0)),
            pl.BlockSpec((1, D), lambda i: (0, 0)),
        ],
        out_specs=pl.BlockSpec((_BLK, D), lambda i: (i, 0)),
        out_shape=jax.ShapeDtypeStruct((N, D), jnp.float32),
    )(p0, p1, dinv, b.reshape(1, D))

    return out


# speedup vs baseline: 38.4930x; 1.4601x over previous
import functools

import jax
import jax.numpy as jnp
from jax import lax
from jax.experimental import pallas as pl
from jax.experimental.pallas import tpu as pltpu
from jax.experimental.pallas import tpu_sc as plsc

N = 10000
E = 320000
D = 128

NC = 2
NS = 16
NW = NC * NS
EW = E // NW
CH = 40
NCHUNK = EW // CH
NB = 5

NPAD = 10240
SEG = NPAD // NS

ROWS_T = N // NS

E_T = E // NS
K1_CH = 2000
K1_NCHUNK = E_T // K1_CH

_mesh = plsc.VectorSubcoreMesh(core_axis_name="c", subcore_axis_name="s")
_sc_params = pltpu.CompilerParams(
    needs_layout_passes=False, use_tc_tiling_on_sc=False
)


@functools.partial(
    pl.kernel,
    mesh=_mesh,
    out_type=jax.ShapeDtypeStruct((NS, SEG), jnp.float32),
    scratch_types=[
        pltpu.VMEM((K1_CH,), jnp.int32),
        pltpu.VMEM((NPAD,), jnp.float32),
        pltpu.VMEM((SEG,), jnp.float32),
        pltpu.VMEM((SEG,), jnp.float32),
        pltpu.VMEM_SHARED((NS, NPAD), jnp.float32),
    ],
    compiler_params=_sc_params,
)
def _deg_kernel(dst_hbm, deg_hbm, idx_v, hist_v, seg_v, acc_v, stage_s):
    cid = lax.axis_index("c")
    sid = lax.axis_index("s")
    ones = jnp.ones((16,), jnp.float32)

    @pl.when(cid == 0)
    def _():
        def zero_body(i, _):
            hist_v[pl.ds(i * 16, 16)] = jnp.zeros((16,), jnp.float32)
            return ()
        lax.fori_loop(0, NPAD // 16, zero_body, ())

        def chunk_body(j, _):
            base = sid * E_T + j * K1_CH
            pltpu.sync_copy(dst_hbm.at[pl.ds(base, K1_CH)], idx_v)

            def vec_body(k, _):
                idx = idx_v[pl.ds(k * 16, 16)]
                plsc.addupdate_scatter(hist_v, [idx], ones)
                return ()
            lax.fori_loop(0, K1_CH // 16, vec_body, ())
            return ()
        lax.fori_loop(0, K1_NCHUNK, chunk_body, ())

        pltpu.sync_copy(hist_v, stage_s.at[sid])

    plsc.subcore_barrier()

    @pl.when(cid == 0)
    def _():
        def rzero(i, _):
            acc_v[pl.ds(i * 16, 16)] = jnp.zeros((16,), jnp.float32)
            return ()
        lax.fori_loop(0, SEG // 16, rzero, ())

        def radd(j, _):
            pltpu.sync_copy(stage_s.at[j, pl.ds(sid * SEG, SEG)], seg_v)

            def vadd(k, _):
                acc_v[pl.ds(k * 16, 16)] = acc_v[pl.ds(k * 16, 16)] + seg_v[pl.ds(k * 16, 16)]
                return ()
            lax.fori_loop(0, SEG // 16, vadd, ())
            return ()
        lax.fori_loop(0, NS, radd, ())

        pltpu.sync_copy(acc_v, deg_hbm.at[sid])


@functools.partial(
    pl.kernel,
    mesh=_mesh,
    out_type=(
        jax.ShapeDtypeStruct((N, D), jnp.float32),
        jax.ShapeDtypeStruct((N, D), jnp.float32),
    ),
    scratch_types=[
        pltpu.VMEM((NCHUNK, CH), jnp.int32),
        pltpu.VMEM((NCHUNK, CH), jnp.int32),
        pltpu.VMEM((NB, CH, D), jnp.float32),
        pltpu.VMEM_SHARED((N, D), jnp.float32),
        pltpu.SemaphoreType.DMA((NB,)),
        pltpu.SemaphoreType.DMA((NB,)),
    ],
    compiler_params=_sc_params,
)
def _msg_kernel(g_hbm, src_hbm, dst_hbm, zeros_hbm, p0_hbm, p1_hbm,
                src_v, dst_v, rows_v, acc_s, gsem, ssem):
    cid = lax.axis_index("c")
    sid = lax.axis_index("s")
    w = sid * NC + cid

    pltpu.sync_copy(src_hbm.at[w], src_v)
    pltpu.sync_copy(dst_hbm.at[w], dst_v)

    base = sid * ROWS_T

    @pl.when(cid == 0)
    def _():
        pltpu.sync_copy(g_hbm.at[pl.ds(base, ROWS_T)], acc_s.at[pl.ds(base, ROWS_T)])

    @pl.when(cid != 0)
    def _():
        pltpu.sync_copy(zeros_hbm.at[pl.ds(base, ROWS_T)], acc_s.at[pl.ds(base, ROWS_T)])

    plsc.subcore_barrier()

    for b in range(NB):
        pltpu.async_copy(g_hbm.at[src_v.at[b]], rows_v.at[b], gsem.at[b])

    def group_body(gi, _):
        for b in range(NB):
            c = gi * NB + b
            pltpu.make_async_copy(
                g_hbm.at[src_v.at[c]], rows_v.at[b], gsem.at[b]
            ).wait()
            pltpu.async_copy(
                rows_v.at[b], acc_s.at[dst_v.at[c]], ssem.at[b], add=True
            )
        for b in range(NB):
            c = gi * NB + b
            pltpu.make_async_copy(
                rows_v.at[b], acc_s.at[dst_v.at[c]], ssem.at[b]
            ).wait()

            @pl.when(c + NB < NCHUNK)
            def _():
                pltpu.async_copy(
                    g_hbm.at[src_v.at[c + NB]], rows_v.at[b], gsem.at[b]
                )
        return ()
    lax.fori_loop(0, NCHUNK // NB, group_body, ())

    plsc.subcore_barrier()

    @pl.when(cid == 0)
    def _():
        pltpu.sync_copy(acc_s.at[pl.ds(base, ROWS_T)], p0_hbm.at[pl.ds(base, ROWS_T)])

    @pl.when(cid != 0)
    def _():
        pltpu.sync_copy(acc_s.at[pl.ds(base, ROWS_T)], p1_hbm.at[pl.ds(base, ROWS_T)])


def _transform_body(x_ref, w_ref, wc_ref, deg_ref, g_ref, dinv_ref):
    x0 = jnp.dot(x_ref[...], w_ref[...], preferred_element_type=jnp.float32)
    h = jnp.dot(x0, wc_ref[...], preferred_element_type=jnp.float32)
    dinv = lax.rsqrt(deg_ref[...] + 1.0)
    g_ref[...] = h * dinv
    dinv_ref[...] = dinv


def _epilogue_body(p0_ref, p1_ref, dinv_ref, b_ref, out_ref):
    s = (p0_ref[...] + p1_ref[...]) * dinv_ref[...] + b_ref[...]
    out_ref[...] = jnp.where(s >= 0, s, 0.2 * s)


_BLK = 1000


def kernel(x, edge_index, W, Wc, b):
    src = edge_index[0].reshape(NW, NCHUNK, CH)
    dst_flat = edge_index[1]
    dst = dst_flat.reshape(NW, NCHUNK, CH)

    deg2 = _deg_kernel(dst_flat)
    deg = deg2.reshape(NPAD, 1)

    g, dinv = pl.pallas_call(
        _transform_body,
        grid=(N // _BLK,),
        in_specs=[
            pl.BlockSpec((_BLK, D), lambda i: (i, 0)),
            pl.BlockSpec((D, D), lambda i: (0, 0)),
            pl.BlockSpec((D, D), lambda i: (0, 0)),
            pl.BlockSpec((_BLK, 1), lambda i: (i, 0)),
        ],
        out_specs=(
            pl.BlockSpec((_BLK, D), lambda i: (i, 0)),
            pl.BlockSpec((_BLK, 1), lambda i: (i, 0)),
        ),
        out_shape=(
            jax.ShapeDtypeStruct((N, D), jnp.float32),
            jax.ShapeDtypeStruct((N, 1), jnp.float32),
        ),
    )(x, W, Wc, deg)

    zeros = jnp.zeros((N, D), jnp.float32)
    p0, p1 = _msg_kernel(g, src, dst, zeros)

    out = pl.pallas_call(
        _epilogue_body,
        grid=(N // _BLK,),
        in_specs=[
            pl.BlockSpec((_BLK, D), lambda i: (i, 0)),
            pl.BlockSpec((_BLK, D), lambda i: (i, 0)),
            pl.BlockSpec((_BLK, 1), lambda i: (i, 0)),
            pl.BlockSpec((1, D), lambda i: (0, 0)),
        ],
        out_specs=pl.BlockSpec((_BLK, D), lambda i: (i, 0)),
        out_shape=jax.ShapeDtypeStruct((N, D), jnp.float32),
    )(p0, p1, dinv, b.reshape(1, D))

    return out


# speedup vs baseline: 39.9500x; 1.0379x over previous
import functools

import jax
import jax.numpy as jnp
from jax import lax
from jax.experimental import pallas as pl
from jax.experimental.pallas import tpu as pltpu
from jax.experimental.pallas import tpu_sc as plsc

N = 10000
E = 320000
D = 128

NC = 2
NS = 16
NW = NC * NS
EW = E // NW
CH = 40
NCHUNK = EW // CH
NB = 6
NTAIL = NCHUNK - (NCHUNK // NB) * NB

NPAD = 10240
SEG = NPAD // NS

ROWS_T = N // NS

E_T = E // (NC * NS)
K1_CH = 2000
K1_NCHUNK = E_T // K1_CH

_mesh = plsc.VectorSubcoreMesh(core_axis_name="c", subcore_axis_name="s")
_sc_params = pltpu.CompilerParams(
    needs_layout_passes=False, use_tc_tiling_on_sc=False
)


@functools.partial(
    pl.kernel,
    mesh=_mesh,
    out_type=jax.ShapeDtypeStruct((NC, NPAD), jnp.float32),
    scratch_types=[
        pltpu.VMEM((K1_CH,), jnp.int32),
        pltpu.VMEM((NPAD,), jnp.float32),
        pltpu.VMEM((SEG,), jnp.float32),
        pltpu.VMEM((SEG,), jnp.float32),
        pltpu.VMEM_SHARED((NS, NPAD), jnp.float32),
    ],
    compiler_params=_sc_params,
)
def _deg_kernel(dst_hbm, deg_hbm, idx_v, hist_v, seg_v, acc_v, stage_s):
    cid = lax.axis_index("c")
    sid = lax.axis_index("s")
    ones = jnp.ones((16,), jnp.float32)

    def zero_body(i, _):
        hist_v[pl.ds(i * 16, 16)] = jnp.zeros((16,), jnp.float32)
        return ()
    lax.fori_loop(0, NPAD // 16, zero_body, ())

    def chunk_body(j, _):
        base = (cid * NS + sid) * E_T + j * K1_CH
        pltpu.sync_copy(dst_hbm.at[pl.ds(base, K1_CH)], idx_v)

        def vec_body(k, _):
            idx = idx_v[pl.ds(k * 16, 16)]
            plsc.addupdate_scatter(hist_v, [idx], ones)
            return ()
        lax.fori_loop(0, K1_CH // 16, vec_body, ())
        return ()
    lax.fori_loop(0, K1_NCHUNK, chunk_body, ())

    pltpu.sync_copy(hist_v, stage_s.at[sid])

    plsc.subcore_barrier()

    def rzero(i, _):
        acc_v[pl.ds(i * 16, 16)] = jnp.zeros((16,), jnp.float32)
        return ()
    lax.fori_loop(0, SEG // 16, rzero, ())

    def radd(j, _):
        pltpu.sync_copy(stage_s.at[j, pl.ds(sid * SEG, SEG)], seg_v)

        def vadd(k, _):
            acc_v[pl.ds(k * 16, 16)] = acc_v[pl.ds(k * 16, 16)] + seg_v[pl.ds(k * 16, 16)]
            return ()
        lax.fori_loop(0, SEG // 16, vadd, ())
        return ()
    lax.fori_loop(0, NS, radd, ())

    pltpu.sync_copy(acc_v, deg_hbm.at[cid, pl.ds(sid * SEG, SEG)])


@functools.partial(
    pl.kernel,
    mesh=_mesh,
    out_type=(
        jax.ShapeDtypeStruct((N, D), jnp.float32),
        jax.ShapeDtypeStruct((N, D), jnp.float32),
    ),
    scratch_types=[
        pltpu.VMEM((NCHUNK, CH), jnp.int32),
        pltpu.VMEM((NCHUNK, CH), jnp.int32),
        pltpu.VMEM((NB, CH, D), jnp.float32),
        pltpu.VMEM_SHARED((N, D), jnp.float32),
        pltpu.SemaphoreType.DMA((NB,)),
        pltpu.SemaphoreType.DMA((NB,)),
    ],
    compiler_params=_sc_params,
)
def _msg_kernel(g_hbm, src_hbm, dst_hbm, zeros_hbm, p0_hbm, p1_hbm,
                src_v, dst_v, rows_v, acc_s, gsem, ssem):
    cid = lax.axis_index("c")
    sid = lax.axis_index("s")
    w = sid * NC + cid

    pltpu.sync_copy(src_hbm.at[w], src_v)
    pltpu.sync_copy(dst_hbm.at[w], dst_v)

    base = sid * ROWS_T

    @pl.when(cid == 0)
    def _():
        pltpu.sync_copy(g_hbm.at[pl.ds(base, ROWS_T)], acc_s.at[pl.ds(base, ROWS_T)])

    @pl.when(cid != 0)
    def _():
        pltpu.sync_copy(zeros_hbm.at[pl.ds(base, ROWS_T)], acc_s.at[pl.ds(base, ROWS_T)])

    plsc.subcore_barrier()

    for b in range(NB):
        pltpu.async_copy(g_hbm.at[src_v.at[b]], rows_v.at[b], gsem.at[b])

    def group_body(gi, _):
        for b in range(NB):
            c = gi * NB + b
            pltpu.make_async_copy(
                g_hbm.at[src_v.at[c]], rows_v.at[b], gsem.at[b]
            ).wait()
            pltpu.async_copy(
                rows_v.at[b], acc_s.at[dst_v.at[c]], ssem.at[b], add=True
            )
        for b in range(NB):
            c = gi * NB + b
            pltpu.make_async_copy(
                rows_v.at[b], acc_s.at[dst_v.at[c]], ssem.at[b]
            ).wait()

            @pl.when(c + NB < NCHUNK)
            def _():
                pltpu.async_copy(
                    g_hbm.at[src_v.at[c + NB]], rows_v.at[b], gsem.at[b]
                )
        return ()
    lax.fori_loop(0, NCHUNK // NB, group_body, ())

    for t in range(NTAIL):
        c = (NCHUNK // NB) * NB + t
        pltpu.make_async_copy(
            g_hbm.at[src_v.at[c]], rows_v.at[t], gsem.at[t]
        ).wait()
        pltpu.async_copy(
            rows_v.at[t], acc_s.at[dst_v.at[c]], ssem.at[t], add=True
        )
    for t in range(NTAIL):
        c = (NCHUNK // NB) * NB + t
        pltpu.make_async_copy(
            rows_v.at[t], acc_s.at[dst_v.at[c]], ssem.at[t]
        ).wait()

    plsc.subcore_barrier()

    @pl.when(cid == 0)
    def _():
        pltpu.sync_copy(acc_s.at[pl.ds(base, ROWS_T)], p0_hbm.at[pl.ds(base, ROWS_T)])

    @pl.when(cid != 0)
    def _():
        pltpu.sync_copy(acc_s.at[pl.ds(base, ROWS_T)], p1_hbm.at[pl.ds(base, ROWS_T)])


def _transform_body(x_ref, w_ref, wc_ref, deg0_ref, deg1_ref, g_ref, dinv_ref):
    x0 = jnp.dot(x_ref[...], w_ref[...], preferred_element_type=jnp.float32)
    h = jnp.dot(x0, wc_ref[...], preferred_element_type=jnp.float32)
    dinv = lax.rsqrt(deg0_ref[...] + deg1_ref[...] + 1.0)
    g_ref[...] = h * dinv
    dinv_ref[...] = dinv


def _epilogue_body(p0_ref, p1_ref, dinv_ref, b_ref, out_ref):
    s = (p0_ref[...] + p1_ref[...]) * dinv_ref[...] + b_ref[...]
    out_ref[...] = jnp.where(s >= 0, s, 0.2 * s)


_BLK = 1000


def kernel(x, edge_index, W, Wc, b):
    src = edge_index[0].reshape(NW, NCHUNK, CH)
    dst_flat = edge_index[1]
    dst = dst_flat.reshape(NW, NCHUNK, CH)

    deg2 = _deg_kernel(dst_flat)
    deg0 = deg2[0].reshape(NPAD, 1)
    deg1 = deg2[1].reshape(NPAD, 1)

    g, dinv = pl.pallas_call(
        _transform_body,
        grid=(N // _BLK,),
        in_specs=[
            pl.BlockSpec((_BLK, D), lambda i: (i, 0)),
            pl.BlockSpec((D, D), lambda i: (0, 0)),
            pl.BlockSpec((D, D), lambda i: (0, 0)),
            pl.BlockSpec((_BLK, 1), lambda i: (i, 0)),
            pl.BlockSpec((_BLK, 1), lambda i: (i, 0)),
        ],
        out_specs=(
            pl.BlockSpec((_BLK, D), lambda i: (i, 0)),
            pl.BlockSpec((_BLK, 1), lambda i: (i, 0)),
        ),
        out_shape=(
            jax.ShapeDtypeStruct((N, D), jnp.float32),
            jax.ShapeDtypeStruct((N, 1), jnp.float32),
        ),
    )(x, W, Wc, deg0, deg1)

    zeros = jnp.zeros((N, D), jnp.float32)
    p0, p1 = _msg_kernel(g, src, dst, zeros)

    out = pl.pallas_call(
        _epilogue_body,
        grid=(N // _BLK,),
        in_specs=[
            pl.BlockSpec((_BLK, D), lambda i: (i, 0)),
            pl.BlockSpec((_BLK, D), lambda i: (i, 0)),
            pl.BlockSpec((_BLK, 1), lambda i: (i, 0)),
            pl.BlockSpec((1, D), lambda i: (0, 0)),
        ],
        out_specs=pl.BlockSpec((_BLK, D), lambda i: (i, 0)),
        out_shape=jax.ShapeDtypeStruct((N, D), jnp.float32),
    )(p0, p1, dinv, b.reshape(1, D))

    return out
